# Initial kernel scaffold; baseline (speedup 1.0000x reference)
import jax
import jax.numpy as jnp
from jax.experimental import pallas as pl


def kernel(user_emb, item_emb, v_feat, t_feat, Wv, bv, Wt, bt, Wu, bu, Wi, bi, edge_index):
    raise NotImplementedError("write your pallas kernel here")


# speedup vs baseline: 14.8402x; 14.8402x over previous
import functools

import jax
import jax.numpy as jnp
from jax import lax
from jax.experimental import pallas as pl
from jax.experimental.pallas import tpu as pltpu
from jax.experimental.pallas import tpu_sc as plsc

NU = 25000
NI = 25000
NN = NU + NI
D = 64
E = 800000

NC = 2
NS = 16
CH = 128
NCHUNK = 391
EPT = NCHUNK * CH
EPAD = EPT * NS
ROWS_PAD = 25088
STRIPE = ROWS_PAD // NS
ABSORB = 25080

DEG_PER_W = (2 * E) // (NC * NS)
CNT_WORDS = 51200

_MESH = plsc.VectorSubcoreMesh(
    core_axis_name="c", subcore_axis_name="s", num_cores=NC, num_subcores=NS)


def _wid():
    return lax.axis_index("s") * NC + lax.axis_index("c")


def _deg_body(allidx_hbm, out_hbm, cnt, idxbuf):
    wid = _wid()
    zeros16 = jnp.zeros((16,), jnp.float32)
    ones16 = jnp.ones((16,), jnp.float32)

    def zero_body(i, c):
        cnt[pl.ds(i * 16, 16)] = zeros16
        return c
    lax.fori_loop(0, CNT_WORDS // 16, zero_body, 0)

    pltpu.sync_copy(allidx_hbm.at[pl.ds(wid * DEG_PER_W, DEG_PER_W)], idxbuf)

    def body(i, c):
        iv = idxbuf[pl.ds(i * 16, 16)]
        plsc.addupdate_scatter(cnt, [iv], ones16)
        return c
    lax.fori_loop(0, DEG_PER_W // 16, body, 0)

    pltpu.sync_copy(cnt, out_hbm.at[wid])


_deg_kernel = functools.partial(
    pl.kernel,
    out_type=jax.ShapeDtypeStruct((NC * NS, CNT_WORDS), jnp.float32),
    mesh=_MESH,
    compiler_params=pltpu.CompilerParams(needs_layout_passes=False),
    scratch_types=[
        pltpu.VMEM((CNT_WORDS,), jnp.float32),
        pltpu.VMEM((DEG_PER_W,), jnp.int32),
    ],
)(_deg_body)


def _seg_body(zeros_hbm,
              s0, d0, t0, s1, d1, t1, s2, d2, t2, s3, d3, t3,
              o0, o1, o2, o3,
              acc, sidx, didx, rows, sem):
    c = lax.axis_index("c")
    sid = lax.axis_index("s")
    r0 = sid * STRIPE

    def run(src, dst, tab, out):
        pltpu.sync_copy(zeros_hbm.at[pl.ds(r0, STRIPE)],
                        acc.at[pl.ds(r0, STRIPE)])
        plsc.subcore_barrier()
        ebase = sid * EPT

        def body(i, carry):
            off = ebase + i * CH
            pltpu.sync_copy(src.at[pl.ds(off, CH)], sidx)
            pltpu.sync_copy(dst.at[pl.ds(off, CH)], didx)
            pltpu.async_copy(tab.at[sidx], rows, sem).wait()
            pltpu.sync_copy(rows, acc.at[didx], add=True)
            return carry
        lax.fori_loop(0, NCHUNK, body, 0)
        plsc.subcore_barrier()
        pltpu.sync_copy(acc.at[pl.ds(r0, STRIPE)], out.at[pl.ds(r0, STRIPE)])

    @pl.when(c == 0)
    def _():
        run(s0, d0, t0, o0)
        run(s1, d1, t1, o1)

    @pl.when(c == 1)
    def _():
        run(s2, d2, t2, o2)
        run(s3, d3, t3, o3)


_OUT4 = tuple(jax.ShapeDtypeStruct((ROWS_PAD, D), jnp.float32) for _ in range(4))

_seg_kernel = functools.partial(
    pl.kernel,
    out_type=_OUT4,
    mesh=_MESH,
    compiler_params=pltpu.CompilerParams(use_tc_tiling_on_sc=False),
    scratch_types=[
        pltpu.MemorySpace.VMEM_SHARED((ROWS_PAD, D), jnp.float32),
        pltpu.VMEM((CH,), jnp.int32),
        pltpu.VMEM((CH,), jnp.int32),
        pltpu.VMEM((CH, D), jnp.float32),
        pltpu.SemaphoreType.DMA,
    ],
)(_seg_body)


def _deg_finish_body(cnt_ref, dinv_ref):
    c = jnp.sum(cnt_ref[...], axis=0)
    dinv_ref[...] = lax.rsqrt(2.0 * c)


def _deg_finish(cnt):
    return pl.pallas_call(
        _deg_finish_body,
        out_shape=jax.ShapeDtypeStruct((CNT_WORDS // 128, 128), jnp.float32),
    )(cnt)


_RB = 1000


def _pre_body(ue_ref, ie_ref, vf_ref, tf_ref, su_ref, si_ref,
              wv_ref, bv_ref, wt_ref, bt_ref,
              vd_ref, td_ref, zu_ref, z0_ref, z1_ref, z2_ref):
    vf = vf_ref[...]
    tf = tf_ref[...]
    vn = vf * lax.rsqrt(jnp.maximum(jnp.sum(vf * vf, axis=1, keepdims=True),
                                    1e-24))
    tn = tf * lax.rsqrt(jnp.maximum(jnp.sum(tf * tf, axis=1, keepdims=True),
                                    1e-24))
    vd = lax.dot_general(vn, wv_ref[...], (((1,), (1,)), ((), ())),
                         preferred_element_type=jnp.float32) + bv_ref[...]
    td = lax.dot_general(tn, wt_ref[...], (((1,), (1,)), ((), ())),
                         preferred_element_type=jnp.float32) + bt_ref[...]
    su = su_ref[...]
    si = si_ref[...]
    vd_ref[...] = vd
    td_ref[...] = td
    zu_ref[...] = su * ue_ref[...]
    z0_ref[...] = si * ie_ref[...]
    z1_ref[...] = si * vd
    z2_ref[...] = si * td


def _pre(user_emb, item_emb, v_feat, t_feat, s_u, s_i, Wv, bv, Wt, bt):
    grid = (NU // _RB,)
    rb = lambda i: (i, 0)
    full = lambda i: (0, 0)
    out_shapes = tuple(jax.ShapeDtypeStruct((NU, D), jnp.float32)
                       for _ in range(6))
    return pl.pallas_call(
        _pre_body,
        grid=grid,
        in_specs=[
            pl.BlockSpec((_RB, D), rb), pl.BlockSpec((_RB, D), rb),
            pl.BlockSpec((_RB, 128), rb), pl.BlockSpec((_RB, 128), rb),
            pl.BlockSpec((_RB, 1), rb), pl.BlockSpec((_RB, 1), rb),
            pl.BlockSpec((D, 128), full), pl.BlockSpec((1, D), full),
            pl.BlockSpec((D, 128), full), pl.BlockSpec((1, D), full),
        ],
        out_specs=tuple(pl.BlockSpec((_RB, D), rb) for _ in range(6)),
        out_shape=out_shapes,
    )(user_emb, item_emb, v_feat, t_feat, s_u, s_i, Wv, bv, Wt, bt)


def _mid_body(u0_ref, u1_ref, u2_ref, ui_ref, su_ref, si_ref,
              z0_ref, z1_ref, z2_ref, zi_ref):
    su2 = jnp.square(su_ref[...])
    si2 = jnp.square(si_ref[...])
    z0_ref[...] = su2 * u0_ref[...]
    z1_ref[...] = su2 * u1_ref[...]
    z2_ref[...] = su2 * u2_ref[...]
    zi_ref[...] = si2 * ui_ref[...]


def _mid(u0, u1, u2, ui, s_u, s_i):
    grid = (NU // _RB,)
    rb = lambda i: (i, 0)
    return pl.pallas_call(
        _mid_body,
        grid=grid,
        in_specs=[pl.BlockSpec((_RB, D), rb)] * 4
        + [pl.BlockSpec((_RB, 1), rb)] * 2,
        out_specs=tuple(pl.BlockSpec((_RB, D), rb) for _ in range(4)),
        out_shape=tuple(jax.ShapeDtypeStruct((NU, D), jnp.float32)
                        for _ in range(4)),
    )(u0, u1, u2, ui, s_u, s_i)


def _final_body(x0_ref, x1_ref, x2_ref, a0_ref, a1_ref, a2_ref,
                b0_ref, b1_ref, b2_ref, s_ref, w_ref, bias_ref, out_ref):
    s = s_ref[...]
    m0 = (x0_ref[...] + s * (a0_ref[...] + b0_ref[...])) * (1.0 / 3.0)
    m1 = (x1_ref[...] + s * (a1_ref[...] + b1_ref[...])) * (1.0 / 3.0)
    m2 = (x2_ref[...] + s * (a2_ref[...] + b2_ref[...])) * (1.0 / 3.0)
    m = jnp.concatenate([m0, m1, m2], axis=1)
    out_ref[...] = lax.dot_general(
        m, w_ref[...], (((1,), (1,)), ((), ())),
        preferred_element_type=jnp.float32) + bias_ref[...]


def _final(xs, u1s, u2s, s, W, b):
    grid = (NU // _RB,)
    rb = lambda i: (i, 0)
    full = lambda i: (0, 0)
    return pl.pallas_call(
        _final_body,
        grid=grid,
        in_specs=[pl.BlockSpec((_RB, D), rb)] * 9
        + [pl.BlockSpec((_RB, 1), rb),
           pl.BlockSpec((D, 3 * D), full), pl.BlockSpec((1, D), full)],
        out_specs=pl.BlockSpec((_RB, D), rb),
        out_shape=jax.ShapeDtypeStruct((NU, D), jnp.float32),
    )(*xs, *u1s, *u2s, s, W, b)


def _pipeline(user_emb, item_emb, v_feat, t_feat, Wv, bv, Wt, bt,
              Wu, bu, Wi, bi, edge_index):
    row = edge_index[0]
    colL = edge_index[1] - NU
    bv = bv.reshape(1, D)
    bt = bt.reshape(1, D)
    bu = bu.reshape(1, D)
    bi = bi.reshape(1, D)

    pad_src = jnp.zeros((EPAD - E,), jnp.int32)
    pad_dst = jnp.full((EPAD - E,), ABSORB, jnp.int32)
    row_src = jnp.concatenate([row, pad_src])
    row_dst = jnp.concatenate([row, pad_dst])
    colL_src = jnp.concatenate([colL, pad_src])
    colL_dst = jnp.concatenate([colL, pad_dst])

    allidx = jnp.concatenate([row, colL + NU])
    cnt = _deg_kernel(allidx)
    dinv = _deg_finish(cnt.reshape(NC * NS, CNT_WORDS // 128, 128))
    s = dinv.reshape(-1)[:NN]
    s_u = s[:NU].reshape(NU, 1)
    s_i = s[NU:].reshape(NI, 1)

    v_dense, t_dense, Zu0, Zi0_0, Zi0_1, Zi0_2 = _pre(
        user_emb, item_emb, v_feat, t_feat, s_u, s_i, Wv, bv, Wt, bt)

    zeros = jnp.zeros((ROWS_PAD, D), jnp.float32)

    U1u0p, U1u1p, U1u2p, U1ip = _seg_kernel(
        zeros,
        colL_src, row_dst, Zi0_0,
        colL_src, row_dst, Zi0_1,
        colL_src, row_dst, Zi0_2,
        row_src, colL_dst, Zu0)
    U1u0, U1u1, U1u2 = U1u0p[:NU], U1u1p[:NU], U1u2p[:NU]
    U1i = U1ip[:NI]

    Z1u0, Z1u1, Z1u2, Z1i = _mid(U1u0, U1u1, U1u2, U1i, s_u, s_i)

    U2up, U2i0p, U2i1p, U2i2p = _seg_kernel(
        zeros,
        colL_src, row_dst, Z1i,
        row_src, colL_dst, Z1u0,
        row_src, colL_dst, Z1u1,
        row_src, colL_dst, Z1u2)
    U2u = U2up[:NU]
    U2i0, U2i1, U2i2 = U2i0p[:NI], U2i1p[:NI], U2i2p[:NI]

    user = _final((user_emb, user_emb, user_emb),
                  (U1u0, U1u1, U1u2),
                  (U2u, U2u, U2u), s_u, Wu, bu)
    item = _final((item_emb, v_dense, t_dense),
                  (U1i, U1i, U1i),
                  (U2i0, U2i1, U2i2), s_i, Wi, bi)
    return (user, item)


def kernel(user_emb, item_emb, v_feat, t_feat, Wv, bv, Wt, bt,
           Wu, bu, Wi, bi, edge_index):
    return _pipeline(user_emb, item_emb, v_feat, t_feat, Wv, bv, Wt, bt,
                     Wu, bu, Wi, bi, edge_index)


# speedup vs baseline: 15.6818x; 1.0567x over previous
import functools

import jax
import jax.numpy as jnp
from jax import lax
from jax.experimental import pallas as pl
from jax.experimental.pallas import tpu as pltpu
from jax.experimental.pallas import tpu_sc as plsc

NU = 25000
NI = 25000
NN = NU + NI
D = 64
E = 800000

NC = 2
NS = 16
CH = 128
KS = 8
NSUPER = 50
NJ = NSUPER // 2
NCHUNK = NSUPER * KS
EPT = NCHUNK * CH
EPAD = EPT * NS
NCROW = EPAD // CH
DH = 32
ROWS_PAD = 25088
STRIPE = ROWS_PAD // NS
ABSORB = 25080

DEG_PER_W = (2 * E) // (NC * NS)
CNT_WORDS = 51200

_MESH = plsc.VectorSubcoreMesh(
    core_axis_name="c", subcore_axis_name="s", num_cores=NC, num_subcores=NS)


def _wid():
    return lax.axis_index("s") * NC + lax.axis_index("c")


def _deg_body(allidx_hbm, out_hbm, cnt, idxbuf):
    wid = _wid()
    zeros16 = jnp.zeros((16,), jnp.float32)
    ones16 = jnp.ones((16,), jnp.float32)

    def zero_body(i, c):
        cnt[pl.ds(i * 16, 16)] = zeros16
        return c
    lax.fori_loop(0, CNT_WORDS // 16, zero_body, 0)

    pltpu.sync_copy(allidx_hbm.at[pl.ds(wid * DEG_PER_W, DEG_PER_W)], idxbuf)

    def body(i, c):
        iv = idxbuf[pl.ds(i * 16, 16)]
        plsc.addupdate_scatter(cnt, [iv], ones16)
        return c
    lax.fori_loop(0, DEG_PER_W // 16, body, 0)

    pltpu.sync_copy(cnt, out_hbm.at[wid])


_deg_kernel = functools.partial(
    pl.kernel,
    out_type=jax.ShapeDtypeStruct((NC * NS, CNT_WORDS), jnp.float32),
    mesh=_MESH,
    compiler_params=pltpu.CompilerParams(needs_layout_passes=False),
    scratch_types=[
        pltpu.VMEM((CNT_WORDS,), jnp.float32),
        pltpu.VMEM((DEG_PER_W,), jnp.int32),
    ],
)(_deg_body)


def _seg_body(zeros_hbm,
              s0, d0, t0a, t0b, s1, d1, t1a, t1b,
              s2, d2, t2a, t2b, s3, d3, t3a, t3b,
              o0a, o0b, o1a, o1b, o2a, o2b, o3a, o3b,
              acc, sidx, didx, rows, gsem, ssem):
    c = lax.axis_index("c")
    sid = lax.axis_index("s")
    r0 = sid * STRIPE

    def run(src, dst, tab, out):
        crow = sid * NCHUNK

        def load_idx(sup, p):
            pltpu.sync_copy(src.at[pl.ds(crow + sup * KS, KS)], sidx.at[p])
            pltpu.sync_copy(dst.at[pl.ds(crow + sup * KS, KS)], didx.at[p])

        def fire_gathers(p):
            for k in range(KS):
                pltpu.async_copy(tab.at[sidx.at[p, k]], rows.at[p, k], gsem)

        def drain_gathers(p):
            for k in range(KS):
                pltpu.make_async_copy(tab.at[sidx.at[p, k]],
                                      rows.at[p, k], gsem).wait()

        def fire_scatters(p):
            for k in range(KS):
                pltpu.async_copy(rows.at[p, k], acc.at[didx.at[p, k]],
                                 ssem, add=True)

        def drain_scatters(p):
            for k in range(KS):
                pltpu.make_async_copy(rows.at[p, k],
                                      acc.at[didx.at[p, k]], ssem).wait()

        load_idx(0, 0)
        fire_gathers(0)
        pltpu.sync_copy(zeros_hbm.at[pl.ds(r0, STRIPE)],
                        acc.at[pl.ds(r0, STRIPE)])
        plsc.subcore_barrier()

        def body(j, carry):
            @pl.when(j > 0)
            def _():
                drain_scatters(1)
            load_idx(2 * j + 1, 1)
            fire_gathers(1)
            drain_gathers(0)
            fire_scatters(0)
            drain_scatters(0)

            @pl.when(j < NJ - 1)
            def _():
                load_idx(2 * j + 2, 0)
                fire_gathers(0)
            drain_gathers(1)
            fire_scatters(1)
            return carry
        lax.fori_loop(0, NJ, body, 0)
        drain_scatters(1)
        plsc.subcore_barrier()
        pltpu.sync_copy(acc.at[pl.ds(r0, STRIPE)], out.at[pl.ds(r0, STRIPE)])

    @pl.when(c == 0)
    def _():
        run(s0, d0, t0a, o0a)
        run(s0, d0, t0b, o0b)
        run(s1, d1, t1a, o1a)
        run(s1, d1, t1b, o1b)

    @pl.when(c == 1)
    def _():
        run(s2, d2, t2a, o2a)
        run(s2, d2, t2b, o2b)
        run(s3, d3, t3a, o3a)
        run(s3, d3, t3b, o3b)


_OUT8 = tuple(jax.ShapeDtypeStruct((ROWS_PAD, DH), jnp.float32)
              for _ in range(8))

_seg_kernel = functools.partial(
    pl.kernel,
    out_type=_OUT8,
    mesh=_MESH,
    compiler_params=pltpu.CompilerParams(use_tc_tiling_on_sc=False),
    scratch_types=[
        pltpu.MemorySpace.VMEM_SHARED((ROWS_PAD, DH), jnp.float32),
        pltpu.VMEM((2, KS, CH), jnp.int32),
        pltpu.VMEM((2, KS, CH), jnp.int32),
        pltpu.VMEM((2, KS, CH, DH), jnp.float32),
        pltpu.SemaphoreType.DMA,
        pltpu.SemaphoreType.DMA,
    ],
)(_seg_body)


def _deg_finish_body(cnt_ref, dinv_ref):
    c = jnp.sum(cnt_ref[...], axis=0)
    dinv_ref[...] = lax.rsqrt(2.0 * c)


def _deg_finish(cnt):
    return pl.pallas_call(
        _deg_finish_body,
        out_shape=jax.ShapeDtypeStruct((CNT_WORDS // 128, 128), jnp.float32),
    )(cnt)


_RB = 1000


def _pre_body(ue_ref, ie_ref, vf_ref, tf_ref, su_ref, si_ref,
              wv_ref, bv_ref, wt_ref, bt_ref,
              vd_ref, td_ref, zu_ref, z0_ref, z1_ref, z2_ref):
    vf = vf_ref[...]
    tf = tf_ref[...]
    vn = vf * lax.rsqrt(jnp.maximum(jnp.sum(vf * vf, axis=1, keepdims=True),
                                    1e-24))
    tn = tf * lax.rsqrt(jnp.maximum(jnp.sum(tf * tf, axis=1, keepdims=True),
                                    1e-24))
    vd = lax.dot_general(vn, wv_ref[...], (((1,), (1,)), ((), ())),
                         preferred_element_type=jnp.float32) + bv_ref[...]
    td = lax.dot_general(tn, wt_ref[...], (((1,), (1,)), ((), ())),
                         preferred_element_type=jnp.float32) + bt_ref[...]
    su = su_ref[...]
    si = si_ref[...]
    vd_ref[...] = vd
    td_ref[...] = td
    zu_ref[...] = su * ue_ref[...]
    z0_ref[...] = si * ie_ref[...]
    z1_ref[...] = si * vd
    z2_ref[...] = si * td


def _pre(user_emb, item_emb, v_feat, t_feat, s_u, s_i, Wv, bv, Wt, bt):
    grid = (NU // _RB,)
    rb = lambda i: (i, 0)
    full = lambda i: (0, 0)
    out_shapes = tuple(jax.ShapeDtypeStruct((NU, D), jnp.float32)
                       for _ in range(6))
    return pl.pallas_call(
        _pre_body,
        grid=grid,
        in_specs=[
            pl.BlockSpec((_RB, D), rb), pl.BlockSpec((_RB, D), rb),
            pl.BlockSpec((_RB, 128), rb), pl.BlockSpec((_RB, 128), rb),
            pl.BlockSpec((_RB, 1), rb), pl.BlockSpec((_RB, 1), rb),
            pl.BlockSpec((D, 128), full), pl.BlockSpec((1, D), full),
            pl.BlockSpec((D, 128), full), pl.BlockSpec((1, D), full),
        ],
        out_specs=tuple(pl.BlockSpec((_RB, D), rb) for _ in range(6)),
        out_shape=out_shapes,
    )(user_emb, item_emb, v_feat, t_feat, s_u, s_i, Wv, bv, Wt, bt)


def _mid_body(u0_ref, u1_ref, u2_ref, ui_ref, su_ref, si_ref,
              z0_ref, z1_ref, z2_ref, zi_ref):
    su2 = jnp.square(su_ref[...])
    si2 = jnp.square(si_ref[...])
    z0_ref[...] = su2 * u0_ref[...]
    z1_ref[...] = su2 * u1_ref[...]
    z2_ref[...] = su2 * u2_ref[...]
    zi_ref[...] = si2 * ui_ref[...]


def _mid(u0, u1, u2, ui, s_u, s_i):
    grid = (NU // _RB,)
    rb = lambda i: (i, 0)
    return pl.pallas_call(
        _mid_body,
        grid=grid,
        in_specs=[pl.BlockSpec((_RB, D), rb)] * 4
        + [pl.BlockSpec((_RB, 1), rb)] * 2,
        out_specs=tuple(pl.BlockSpec((_RB, D), rb) for _ in range(4)),
        out_shape=tuple(jax.ShapeDtypeStruct((NU, D), jnp.float32)
                        for _ in range(4)),
    )(u0, u1, u2, ui, s_u, s_i)


def _final_body(x0_ref, x1_ref, x2_ref, a0_ref, a1_ref, a2_ref,
                b0_ref, b1_ref, b2_ref, s_ref, w_ref, bias_ref, out_ref):
    s = s_ref[...]
    m0 = (x0_ref[...] + s * (a0_ref[...] + b0_ref[...])) * (1.0 / 3.0)
    m1 = (x1_ref[...] + s * (a1_ref[...] + b1_ref[...])) * (1.0 / 3.0)
    m2 = (x2_ref[...] + s * (a2_ref[...] + b2_ref[...])) * (1.0 / 3.0)
    m = jnp.concatenate([m0, m1, m2], axis=1)
    out_ref[...] = lax.dot_general(
        m, w_ref[...], (((1,), (1,)), ((), ())),
        preferred_element_type=jnp.float32) + bias_ref[...]


def _final(xs, u1s, u2s, s, W, b):
    grid = (NU // _RB,)
    rb = lambda i: (i, 0)
    full = lambda i: (0, 0)
    return pl.pallas_call(
        _final_body,
        grid=grid,
        in_specs=[pl.BlockSpec((_RB, D), rb)] * 9
        + [pl.BlockSpec((_RB, 1), rb),
           pl.BlockSpec((D, 3 * D), full), pl.BlockSpec((1, D), full)],
        out_specs=pl.BlockSpec((_RB, D), rb),
        out_shape=jax.ShapeDtypeStruct((NU, D), jnp.float32),
    )(*xs, *u1s, *u2s, s, W, b)


def _pipeline(user_emb, item_emb, v_feat, t_feat, Wv, bv, Wt, bt,
              Wu, bu, Wi, bi, edge_index):
    row = edge_index[0]
    colL = edge_index[1] - NU
    bv = bv.reshape(1, D)
    bt = bt.reshape(1, D)
    bu = bu.reshape(1, D)
    bi = bi.reshape(1, D)

    pad_src = jnp.zeros((EPAD - E,), jnp.int32)
    pad_dst = jnp.full((EPAD - E,), ABSORB, jnp.int32)
    row_src = jnp.concatenate([row, pad_src]).reshape(NCROW, CH)
    row_dst = jnp.concatenate([row, pad_dst]).reshape(NCROW, CH)
    colL_src = jnp.concatenate([colL, pad_src]).reshape(NCROW, CH)
    colL_dst = jnp.concatenate([colL, pad_dst]).reshape(NCROW, CH)

    allidx = jnp.concatenate([row, colL + NU])
    cnt = _deg_kernel(allidx)
    dinv = _deg_finish(cnt.reshape(NC * NS, CNT_WORDS // 128, 128))
    s = dinv.reshape(-1)[:NN]
    s_u = s[:NU].reshape(NU, 1)
    s_i = s[NU:].reshape(NI, 1)

    v_dense, t_dense, Zu0, Zi0_0, Zi0_1, Zi0_2 = _pre(
        user_emb, item_emb, v_feat, t_feat, s_u, s_i, Wv, bv, Wt, bt)

    zeros = jnp.zeros((ROWS_PAD, DH), jnp.float32)

    def seg4(sd0, t0, sd1, t1, sd2, t2, sd3, t3):
        halves = _seg_kernel(
            zeros,
            sd0[0], sd0[1], t0[:, :DH], t0[:, DH:],
            sd1[0], sd1[1], t1[:, :DH], t1[:, DH:],
            sd2[0], sd2[1], t2[:, :DH], t2[:, DH:],
            sd3[0], sd3[1], t3[:, :DH], t3[:, DH:])
        return tuple(
            jnp.concatenate([halves[2 * i][:NU], halves[2 * i + 1][:NU]],
                            axis=1)
            for i in range(4))

    iu = (colL_src, row_dst)
    ui = (row_src, colL_dst)

    U1u0, U1u1, U1u2, U1i = seg4(iu, Zi0_0, iu, Zi0_1, iu, Zi0_2, ui, Zu0)

    Z1u0, Z1u1, Z1u2, Z1i = _mid(U1u0, U1u1, U1u2, U1i, s_u, s_i)

    U2u, U2i0, U2i1, U2i2 = seg4(iu, Z1i, ui, Z1u0, ui, Z1u1, ui, Z1u2)

    user = _final((user_emb, user_emb, user_emb),
                  (U1u0, U1u1, U1u2),
                  (U2u, U2u, U2u), s_u, Wu, bu)
    item = _final((item_emb, v_dense, t_dense),
                  (U1i, U1i, U1i),
                  (U2i0, U2i1, U2i2), s_i, Wi, bi)
    return (user, item)


def kernel(user_emb, item_emb, v_feat, t_feat, Wv, bv, Wt, bt,
           Wu, bu, Wi, bi, edge_index):
    return _pipeline(user_emb, item_emb, v_feat, t_feat, Wv, bv, Wt, bt,
                     Wu, bu, Wi, bi, edge_index)
